# baseline (speedup 1.0000x reference)
import functools

import jax
import jax.numpy as jnp
from jax import lax
from jax.experimental import pallas as pl
from jax.experimental.pallas import tpu as pltpu
from jax.experimental.pallas import tpu_sc as plsc

_B, _D, _K, _N, _H, _M = 1024, 128, 8, 512, 512, 8192
_NK = _N * _K

_NC, _NS = 2, 16
_NW = _NC * _NS
_RPW = _B // _NW
_CH = 8
_NCHUNK = _RPW // _CH

_BBA = 128
_BBU = 256
_BBL = 256


def _nn_body(z_ref, bz_ref, out_ref):
    z = z_ref[...]
    bz = bz_ref[...]
    s = lax.dot_general(z, bz, (((1,), (1,)), ((), ())),
                        preferred_element_type=jnp.float32)
    zn = jnp.sum(z * z, axis=1, keepdims=True)
    bn = lax.dot_general(jnp.ones((1, _D), jnp.float32), bz * bz,
                         (((1,), (1,)), ((), ())),
                         preferred_element_type=jnp.float32)
    d2 = zn - 2.0 * s + bn
    mv = jnp.min(d2, axis=1, keepdims=True)
    ii = lax.broadcasted_iota(jnp.int32, d2.shape, 1)
    idx = jnp.min(jnp.where(d2 <= mv, ii, jnp.int32(_M)), axis=1)
    out_ref[...] = idx.reshape(1, 1, _BBA)


def _nn_call(z, bz):
    nblk = _B // _BBA
    return pl.pallas_call(
        _nn_body,
        grid=(nblk,),
        in_specs=[
            pl.BlockSpec((_BBA, _D), lambda i: (i, 0)),
            pl.BlockSpec((_M, _D), lambda i: (0, 0)),
        ],
        out_specs=pl.BlockSpec((1, 1, _BBA), lambda i: (i, 0, 0)),
        out_shape=jax.ShapeDtypeStruct((nblk, 1, _BBA), jnp.int32),
    )(z, bz)


def _make_sc_gather():
    mesh = plsc.VectorSubcoreMesh(core_axis_name="c", subcore_axis_name="s")

    @functools.partial(
        pl.kernel,
        out_type=jax.ShapeDtypeStruct((_B, _NK), jnp.float32),
        mesh=mesh,
        compiler_params=pltpu.CompilerParams(needs_layout_passes=False),
        scratch_types=[
            pltpu.VMEM((_RPW,), jnp.int32),
            pltpu.VMEM((_CH, _NK), jnp.float32),
            pltpu.VMEM((_CH, _NK), jnp.float32),
            pltpu.SemaphoreType.DMA,
        ],
    )
    def gk(table_hbm, idx_hbm, out_hbm, idx_v, rows_v, rowst_v, sem):
        wid = lax.axis_index("s") * _NC + lax.axis_index("c")
        base = wid * _RPW
        pltpu.sync_copy(idx_hbm.at[wid], idx_v)
        pat = lax.iota(jnp.int32, 16) * 8
        for c in range(_NCHUNK):
            pltpu.async_copy(
                table_hbm.at[idx_v.at[pl.ds(c * _CH, _CH)]], rows_v, sem
            ).wait()
            for r in range(_CH):
                rsplat = jnp.full((16,), r, jnp.int32)

                def body(c2, carry):
                    for k in range(_K):
                        iv = pat + (c2 * 128 + k)
                        v = plsc.load_gather(rows_v, [rsplat, iv])
                        rowst_v[r, pl.ds(k * _N + c2 * 16, 16)] = v
                    return carry

                lax.fori_loop(0, _NK // 128, body, 0)
            pltpu.sync_copy(rowst_v, out_hbm.at[pl.ds(base + c * _CH, _CH)])

    return gk


def _u_body(z_ref, w1_ref, b1_ref, vt_ref, w2_ref, out_ref):
    z = z_ref[...]
    w1 = w1_ref[...]
    b1 = b1_ref[...]
    vt = vt_ref[...]
    w2 = w2_ref[...]
    a = lax.dot_general(z, w1, (((1,), (0,)), ((), ())),
                        preferred_element_type=jnp.float32) + b1
    th = jnp.tanh(a)
    s = 1.0 - th * th
    for k in range(_K):
        pk = lax.dot_general(vt[k:k + 1, :], w1, (((1,), (0,)), ((), ())),
                             preferred_element_type=jnp.float32)
        tk = s * pk
        uk = lax.dot_general(tk, w2, (((1,), (0,)), ((), ())),
                             preferred_element_type=jnp.float32)
        out_ref[:, k * _N:(k + 1) * _N] = uk


def _u_call(z, w1, b1, vt, w2):
    nblk = _B // _BBU
    return pl.pallas_call(
        _u_body,
        grid=(nblk,),
        in_specs=[
            pl.BlockSpec((_BBU, _D), lambda i: (i, 0)),
            pl.BlockSpec((_D, _H), lambda i: (0, 0)),
            pl.BlockSpec((1, _H), lambda i: (0, 0)),
            pl.BlockSpec((_K, _D), lambda i: (0, 0)),
            pl.BlockSpec((_H, _N), lambda i: (0, 0)),
        ],
        out_specs=pl.BlockSpec((_BBU, _NK), lambda i: (i, 0)),
        out_shape=jax.ShapeDtypeStruct((_B, _NK), jnp.float32),
    )(z, w1, b1, vt, w2)


def _loss_body(q_ref, u_ref, out_ref, acc_ref):
    i = pl.program_id(0)
    q = q_ref[...]
    u = u_ref[...]
    su = jnp.sum(u * u)
    uls = [u[:, l * _N:(l + 1) * _N] for l in range(_K)]
    qt = 0.0
    for k in range(_K):
        qk = q[:, k * _N:(k + 1) * _N]
        for l in range(_K):
            r = jnp.sum(qk * uls[l], axis=1, keepdims=True)
            qt = qt + jnp.sum(r * r)

    @pl.when(i == 0)
    def _():
        acc_ref[0] = 0.0

    acc_ref[0] += su - qt

    @pl.when(i == pl.num_programs(0) - 1)
    def _():
        out_ref[0] = acc_ref[0] / jnp.float32(_B)


def _loss_call(qcat, ucat):
    nblk = _B // _BBL
    return pl.pallas_call(
        _loss_body,
        grid=(nblk,),
        in_specs=[
            pl.BlockSpec((_BBL, _NK), lambda i: (i, 0)),
            pl.BlockSpec((_BBL, _NK), lambda i: (i, 0)),
        ],
        out_specs=pl.BlockSpec(memory_space=pltpu.SMEM),
        out_shape=jax.ShapeDtypeStruct((1,), jnp.float32),
        scratch_shapes=[pltpu.SMEM((1,), jnp.float32)],
    )(qcat, ucat)


def kernel(z_prior, V, bank_z, bank_Q, W1, b1, W2, b2):
    idx3 = _nn_call(z_prior, bank_z)
    idx = idx3.reshape(_NW, _RPW)
    table = bank_Q.reshape(_M, _NK)
    qcat = _make_sc_gather()(table, idx)
    ucat = _u_call(z_prior, W1, b1.reshape(1, _H), V.T, W2)
    loss = _loss_call(qcat, ucat)
    return loss.reshape(())


# speedup vs baseline: 1.9903x; 1.9903x over previous
import functools

import jax
import jax.numpy as jnp
from jax import lax
from jax.experimental import pallas as pl
from jax.experimental.pallas import tpu as pltpu
from jax.experimental.pallas import tpu_sc as plsc

_B, _D, _K, _N, _H, _M = 1024, 128, 8, 512, 512, 8192
_NK = _N * _K

_NC, _NS = 2, 16
_NW = _NC * _NS
_RPW = _B // _NW
_CH = 8
_NCHUNK = _RPW // _CH

_BBA = 128
_BBU = 256
_BBL = 256


def _nn_body(z_ref, bz_ref, out_ref):
    z = z_ref[...]
    bz = bz_ref[...]
    s = lax.dot_general(z, bz, (((1,), (1,)), ((), ())),
                        preferred_element_type=jnp.float32)
    zn = jnp.sum(z * z, axis=1, keepdims=True)
    bn = lax.dot_general(jnp.ones((1, _D), jnp.float32), bz * bz,
                         (((1,), (1,)), ((), ())),
                         preferred_element_type=jnp.float32)
    d2 = zn - 2.0 * s + bn
    mv = jnp.min(d2, axis=1, keepdims=True)
    ii = lax.broadcasted_iota(jnp.int32, d2.shape, 1)
    idx = jnp.min(jnp.where(d2 <= mv, ii, jnp.int32(_M)), axis=1)
    out_ref[...] = idx.reshape(1, 1, _BBA)


def _nn_call(z, bz):
    nblk = _B // _BBA
    return pl.pallas_call(
        _nn_body,
        grid=(nblk,),
        in_specs=[
            pl.BlockSpec((_BBA, _D), lambda i: (i, 0)),
            pl.BlockSpec((_M, _D), lambda i: (0, 0)),
        ],
        out_specs=pl.BlockSpec((1, 1, _BBA), lambda i: (i, 0, 0)),
        out_shape=jax.ShapeDtypeStruct((nblk, 1, _BBA), jnp.int32),
    )(z, bz)


def _make_sc_gather():
    mesh = plsc.VectorSubcoreMesh(core_axis_name="c", subcore_axis_name="s")

    @functools.partial(
        pl.kernel,
        out_type=jax.ShapeDtypeStruct((_B, _NK), jnp.float32),
        mesh=mesh,
        compiler_params=pltpu.CompilerParams(needs_layout_passes=False),
        scratch_types=[
            pltpu.VMEM((_RPW,), jnp.int32),
            pltpu.VMEM((_CH, _NK), jnp.float32),
            pltpu.VMEM((_CH, _NK), jnp.float32),
            pltpu.SemaphoreType.DMA,
            pltpu.SemaphoreType.DMA,
        ],
    )
    def gk(table_hbm, idx_hbm, out_hbm, idx_v, buf0, buf1, sem0, sem1):
        wid = lax.axis_index("s") * _NC + lax.axis_index("c")
        base = wid * _RPW
        pltpu.sync_copy(idx_hbm.at[wid], idx_v)
        bufs, sems = [buf0, buf1], [sem0, sem1]
        handles = [None, None]
        handles[0] = pltpu.async_copy(
            table_hbm.at[idx_v.at[pl.ds(0, _CH)]], buf0, sem0)
        for c in range(_NCHUNK):
            nc = c + 1
            if nc < _NCHUNK:
                handles[nc % 2] = pltpu.async_copy(
                    table_hbm.at[idx_v.at[pl.ds(nc * _CH, _CH)]],
                    bufs[nc % 2], sems[nc % 2])
            handles[c % 2].wait()
            pltpu.sync_copy(bufs[c % 2],
                            out_hbm.at[pl.ds(base + c * _CH, _CH)])

    return gk


def _u_body(z_ref, w1_ref, b1_ref, vt_ref, w2_ref, out_ref):
    z = z_ref[...]
    w1 = w1_ref[...]
    b1 = b1_ref[...]
    vt = vt_ref[...]
    w2 = w2_ref[...]
    a = lax.dot_general(z, w1, (((1,), (0,)), ((), ())),
                        preferred_element_type=jnp.float32) + b1
    th = jnp.tanh(a)
    s = 1.0 - th * th
    for k in range(_K):
        pk = lax.dot_general(vt[k:k + 1, :], w1, (((1,), (0,)), ((), ())),
                             preferred_element_type=jnp.float32)
        tk = s * pk
        uk = lax.dot_general(tk, w2, (((1,), (0,)), ((), ())),
                             preferred_element_type=jnp.float32)
        out_ref[:, k * _N:(k + 1) * _N] = uk


def _u_call(z, w1, b1, vt, w2):
    nblk = _B // _BBU
    return pl.pallas_call(
        _u_body,
        grid=(nblk,),
        in_specs=[
            pl.BlockSpec((_BBU, _D), lambda i: (i, 0)),
            pl.BlockSpec((_D, _H), lambda i: (0, 0)),
            pl.BlockSpec((1, _H), lambda i: (0, 0)),
            pl.BlockSpec((_K, _D), lambda i: (0, 0)),
            pl.BlockSpec((_H, _N), lambda i: (0, 0)),
        ],
        out_specs=pl.BlockSpec((_BBU, _NK), lambda i: (i, 0)),
        out_shape=jax.ShapeDtypeStruct((_B, _NK), jnp.float32),
    )(z, w1, b1, vt, w2)


def _loss_body(q_ref, u_ref, out_ref, acc_ref):
    i = pl.program_id(0)
    q = q_ref[...]
    u = u_ref[...]
    su = jnp.sum(u * u)
    uls = [u[:, l * _N:(l + 1) * _N] for l in range(_K)]
    qt = 0.0
    for k in range(_K):
        qk = q[:, k * _N:(k + 1) * _N]
        for l in range(_K):
            r = jnp.sum(qk * uls[l], axis=1, keepdims=True)
            qt = qt + jnp.sum(r * r)

    @pl.when(i == 0)
    def _():
        acc_ref[0] = 0.0

    acc_ref[0] += su - qt

    @pl.when(i == pl.num_programs(0) - 1)
    def _():
        out_ref[0] = acc_ref[0] / jnp.float32(_B)


def _loss_call(qcat, ucat):
    nblk = _B // _BBL
    return pl.pallas_call(
        _loss_body,
        grid=(nblk,),
        in_specs=[
            pl.BlockSpec((_BBL, _NK), lambda i: (i, 0)),
            pl.BlockSpec((_BBL, _NK), lambda i: (i, 0)),
        ],
        out_specs=pl.BlockSpec(memory_space=pltpu.SMEM),
        out_shape=jax.ShapeDtypeStruct((1,), jnp.float32),
        scratch_shapes=[pltpu.SMEM((1,), jnp.float32)],
    )(qcat, ucat)


def kernel(z_prior, V, bank_z, bank_Q, W1, b1, W2, b2):
    idx3 = _nn_call(z_prior, bank_z)
    idx = idx3.reshape(_NW, _RPW)
    table = bank_Q.transpose(0, 2, 1).reshape(_M, _NK)
    qcat = _make_sc_gather()(table, idx)
    ucat = _u_call(z_prior, W1, b1.reshape(1, _H), V.T, W2)
    loss = _loss_call(qcat, ucat)
    return loss.reshape(())


# speedup vs baseline: 3.0443x; 1.5296x over previous
import functools

import jax
import jax.numpy as jnp
from jax import lax
from jax.experimental import pallas as pl
from jax.experimental.pallas import tpu as pltpu
from jax.experimental.pallas import tpu_sc as plsc

_B, _D, _K, _N, _H, _M = 1024, 128, 8, 512, 512, 8192
_NK = _N * _K

_NC, _NS = 2, 16
_NW = _NC * _NS
_RPW = _B // _NW
_CH = 8
_NCHUNK = _RPW // _CH

_BBA = 128
_BBU = 256
_BBL = 256


def _nn_body(z_ref, bz_ref, out_ref):
    z = z_ref[...]
    bz = bz_ref[...]
    s = lax.dot_general(z, bz, (((1,), (1,)), ((), ())),
                        preferred_element_type=jnp.float32)
    zn = jnp.sum(z * z, axis=1, keepdims=True)
    bn = lax.dot_general(jnp.ones((1, _D), jnp.float32), bz * bz,
                         (((1,), (1,)), ((), ())),
                         preferred_element_type=jnp.float32)
    d2 = zn - 2.0 * s + bn
    mv = jnp.min(d2, axis=1, keepdims=True)
    ii = lax.broadcasted_iota(jnp.int32, d2.shape, 1)
    idx = jnp.min(jnp.where(d2 <= mv, ii, jnp.int32(_M)), axis=1)
    out_ref[...] = idx.reshape(1, 1, _BBA)


def _nn_call(z, bz):
    nblk = _B // _BBA
    return pl.pallas_call(
        _nn_body,
        grid=(nblk,),
        in_specs=[
            pl.BlockSpec((_BBA, _D), lambda i: (i, 0)),
            pl.BlockSpec((_M, _D), lambda i: (0, 0)),
        ],
        out_specs=pl.BlockSpec((1, 1, _BBA), lambda i: (i, 0, 0)),
        out_shape=jax.ShapeDtypeStruct((nblk, 1, _BBA), jnp.int32),
    )(z, bz)


def _make_sc_gather():
    mesh = plsc.VectorSubcoreMesh(core_axis_name="c", subcore_axis_name="s")

    @functools.partial(
        pl.kernel,
        out_type=jax.ShapeDtypeStruct((_B, _K, _N), jnp.float32),
        mesh=mesh,
        compiler_params=pltpu.CompilerParams(needs_layout_passes=False),
        scratch_types=[
            pltpu.VMEM((_RPW,), jnp.int32),
            pltpu.VMEM((_CH, _K, _N), jnp.float32),
            pltpu.VMEM((_CH, _K, _N), jnp.float32),
            pltpu.SemaphoreType.DMA,
            pltpu.SemaphoreType.DMA,
        ],
    )
    def gk(table_hbm, idx_hbm, out_hbm, idx_v, buf0, buf1, sem0, sem1):
        wid = lax.axis_index("s") * _NC + lax.axis_index("c")
        base = wid * _RPW
        pltpu.sync_copy(idx_hbm.at[wid], idx_v)
        bufs, sems = [buf0, buf1], [sem0, sem1]
        handles = [None, None]
        handles[0] = pltpu.async_copy(
            table_hbm.at[idx_v.at[pl.ds(0, _CH)]], buf0, sem0)
        for c in range(_NCHUNK):
            nc = c + 1
            if nc < _NCHUNK:
                handles[nc % 2] = pltpu.async_copy(
                    table_hbm.at[idx_v.at[pl.ds(nc * _CH, _CH)]],
                    bufs[nc % 2], sems[nc % 2])
            handles[c % 2].wait()
            pltpu.sync_copy(bufs[c % 2],
                            out_hbm.at[pl.ds(base + c * _CH, _CH)])

    return gk


def _u_body(z_ref, w1_ref, b1_ref, vt_ref, w2_ref, out_ref):
    z = z_ref[...]
    w1 = w1_ref[...]
    b1 = b1_ref[...]
    vt = vt_ref[...]
    w2 = w2_ref[...]
    a = lax.dot_general(z, w1, (((1,), (0,)), ((), ())),
                        preferred_element_type=jnp.float32) + b1
    th = jnp.tanh(a)
    s = 1.0 - th * th
    for k in range(_K):
        pk = lax.dot_general(vt[k:k + 1, :], w1, (((1,), (0,)), ((), ())),
                             preferred_element_type=jnp.float32)
        tk = s * pk
        uk = lax.dot_general(tk, w2, (((1,), (0,)), ((), ())),
                             preferred_element_type=jnp.float32)
        out_ref[:, k * _N:(k + 1) * _N] = uk


def _u_call(z, w1, b1, vt, w2):
    nblk = _B // _BBU
    return pl.pallas_call(
        _u_body,
        grid=(nblk,),
        in_specs=[
            pl.BlockSpec((_BBU, _D), lambda i: (i, 0)),
            pl.BlockSpec((_D, _H), lambda i: (0, 0)),
            pl.BlockSpec((1, _H), lambda i: (0, 0)),
            pl.BlockSpec((_K, _D), lambda i: (0, 0)),
            pl.BlockSpec((_H, _N), lambda i: (0, 0)),
        ],
        out_specs=pl.BlockSpec((_BBU, _NK), lambda i: (i, 0)),
        out_shape=jax.ShapeDtypeStruct((_B, _NK), jnp.float32),
    )(z, w1, b1, vt, w2)


def _loss_body(q_ref, u_ref, out_ref, acc_ref):
    i = pl.program_id(0)
    q = q_ref[...]
    u = u_ref[...]
    su = jnp.sum(u * u)
    uls = [u[:, l * _N:(l + 1) * _N] for l in range(_K)]
    qt = 0.0
    for k in range(_K):
        qk = q[:, k * _N:(k + 1) * _N]
        for l in range(_K):
            r = jnp.sum(qk * uls[l], axis=1, keepdims=True)
            qt = qt + jnp.sum(r * r)

    @pl.when(i == 0)
    def _():
        acc_ref[0] = 0.0

    acc_ref[0] += su - qt

    @pl.when(i == pl.num_programs(0) - 1)
    def _():
        out_ref[0] = acc_ref[0] / jnp.float32(_B)


def _loss_call(qcat, ucat):
    nblk = _B // _BBL
    return pl.pallas_call(
        _loss_body,
        grid=(nblk,),
        in_specs=[
            pl.BlockSpec((_BBL, _NK), lambda i: (i, 0)),
            pl.BlockSpec((_BBL, _NK), lambda i: (i, 0)),
        ],
        out_specs=pl.BlockSpec(memory_space=pltpu.SMEM),
        out_shape=jax.ShapeDtypeStruct((1,), jnp.float32),
        scratch_shapes=[pltpu.SMEM((1,), jnp.float32)],
    )(qcat, ucat)


def kernel(z_prior, V, bank_z, bank_Q, W1, b1, W2, b2):
    idx3 = _nn_call(z_prior, bank_z)
    idx = idx3.reshape(_NW, _RPW)
    table = bank_Q.transpose(0, 2, 1)
    qcat = _make_sc_gather()(table, idx).reshape(_B, _NK)
    ucat = _u_call(z_prior, W1, b1.reshape(1, _H), V.T, W2)
    loss = _loss_call(qcat, ucat)
    return loss.reshape(())
